# Initial kernel scaffold; baseline (speedup 1.0000x reference)
import jax
import jax.numpy as jnp
from jax.experimental import pallas as pl


def kernel(indices, table):
    raise NotImplementedError("write your pallas kernel here")


# speedup vs baseline: 1.5685x; 1.5685x over previous
import jax
import jax.numpy as jnp
from jax import lax
from jax.experimental import pallas as pl
from jax.experimental.pallas import tpu as pltpu
from jax.experimental.pallas import tpu_sc as plsc

BATCH = 16384
N_FEATURES = 26
DIM = 32

NC, NS = 2, 16
NW = NC * NS
B = BATCH * N_FEATURES
B_PER_W = B // NW
CH = 128
CHUNKS_PER_W = B_PER_W // CH
SUP = 8
NSUP = CHUNKS_PER_W // SUP
SUP_ROWS = SUP * CH


def _gather_kernel(table_hbm, idx_hbm, out_hbm, idx_v, rows_v, sem):
    wid = lax.axis_index("s") * NC + lax.axis_index("c")
    pltpu.sync_copy(idx_hbm.at[pl.ds(wid * CHUNKS_PER_W, CHUNKS_PER_W)], idx_v)

    def body(s, carry):
        cps = []
        for c in range(SUP):
            cp = pltpu.async_copy(
                table_hbm.at[idx_v.at[s * SUP + c]],
                rows_v.at[pl.ds(c * CH, CH)],
                sem,
            )
            cps.append(cp)
        for cp in cps:
            cp.wait()
        pltpu.sync_copy(
            rows_v, out_hbm.at[pl.ds(wid * B_PER_W + s * SUP_ROWS, SUP_ROWS)]
        )
        return carry

    lax.fori_loop(0, NSUP, body, 0)


def kernel(indices, table):
    idx2d = indices.reshape(NW * CHUNKS_PER_W, CH)
    mesh = plsc.VectorSubcoreMesh(core_axis_name="c", subcore_axis_name="s")
    out = pl.kernel(
        _gather_kernel,
        out_type=jax.ShapeDtypeStruct((B, DIM), jnp.float32),
        mesh=mesh,
        scratch_types=[
            pltpu.VMEM((CHUNKS_PER_W, CH), jnp.int32),
            pltpu.VMEM((SUP_ROWS, DIM), jnp.float32),
            pltpu.SemaphoreType.DMA,
        ],
        compiler_params=pltpu.CompilerParams(use_tc_tiling_on_sc=False),
    )(table, idx2d)
    return out.reshape(BATCH, N_FEATURES, DIM)


# speedup vs baseline: 1.5899x; 1.0136x over previous
import jax
import jax.numpy as jnp
from jax import lax
from jax.experimental import pallas as pl
from jax.experimental.pallas import tpu as pltpu
from jax.experimental.pallas import tpu_sc as plsc

BATCH = 16384
N_FEATURES = 26
DIM = 32

NC, NS = 2, 16
NW = NC * NS
B = BATCH * N_FEATURES
B_PER_W = B // NW
CH = 128
CHUNKS_PER_W = B_PER_W // CH
SUP = 8
NSUP = CHUNKS_PER_W // SUP
SUP_ROWS = SUP * CH


def _gather_kernel(table_hbm, idx_hbm, out_hbm, idx_v, buf, gsem, ssem):
    wid = lax.axis_index("s") * NC + lax.axis_index("c")
    base = wid * B_PER_W
    pltpu.sync_copy(idx_hbm.at[pl.ds(wid * CHUNKS_PER_W, CHUNKS_PER_W)], idx_v)

    def start_gather(s, slot):
        for c in range(SUP):
            pltpu.async_copy(
                table_hbm.at[idx_v.at[s * SUP + c]],
                buf.at[slot, pl.ds(c * CH, CH)],
                gsem.at[slot],
            )

    def wait_gather(slot):
        pltpu.make_async_copy(
            table_hbm.at[pl.ds(0, SUP_ROWS)], buf.at[slot], gsem.at[slot]
        ).wait()

    def start_store(s, slot):
        pltpu.async_copy(
            buf.at[slot], out_hbm.at[pl.ds(base + s * SUP_ROWS, SUP_ROWS)],
            ssem.at[slot],
        )

    def wait_store(slot):
        pltpu.make_async_copy(
            buf.at[slot], out_hbm.at[pl.ds(base, SUP_ROWS)], ssem.at[slot]
        ).wait()

    start_gather(0, 0)

    def body(s, carry):
        slot = s & 1
        prev = 1 - slot

        @pl.when(s >= 2)
        def _():
            wait_store(slot)

        start_gather(s, slot)
        wait_gather(prev)
        start_store(s - 1, prev)
        return carry

    lax.fori_loop(1, NSUP, body, 0)

    last = (NSUP - 1) & 1
    wait_gather(last)
    start_store(NSUP - 1, last)
    wait_store(1 - last)
    wait_store(last)


def kernel(indices, table):
    idx2d = indices.reshape(NW * CHUNKS_PER_W, CH)
    mesh = plsc.VectorSubcoreMesh(core_axis_name="c", subcore_axis_name="s")
    out = pl.kernel(
        _gather_kernel,
        out_type=jax.ShapeDtypeStruct((B, DIM), jnp.float32),
        mesh=mesh,
        scratch_types=[
            pltpu.VMEM((CHUNKS_PER_W, CH), jnp.int32),
            pltpu.VMEM((2, SUP_ROWS, DIM), jnp.float32),
            pltpu.SemaphoreType.DMA((2,)),
            pltpu.SemaphoreType.DMA((2,)),
        ],
        compiler_params=pltpu.CompilerParams(use_tc_tiling_on_sc=False),
    )(table, idx2d)
    return out.reshape(BATCH, N_FEATURES, DIM)
